# baseline (speedup 1.0000x reference)
import functools

import jax
import jax.numpy as jnp
from jax import lax
from jax.experimental import pallas as pl
from jax.experimental.pallas import tpu as pltpu
from jax.experimental.pallas import tpu_sc as plsc

_N = 10000
_E = 320000
_D = 128
_H = 128
_NE = 8
_NG = 64

_NW = 32
_NSUB = 16
_CH = 128
_NCH = 80
_E_PAD = _NW * _NCH * _CH
_N_PAD = 10240
_RPW = _N_PAD // _NSUB
_BLK = 1024
_GRID = _N_PAD // _BLK
_EB = 6400
_EGRID = _E // _EB
_NHI = _N_PAD // 128


@functools.cache
def _mesh():
    return plsc.VectorSubcoreMesh(core_axis_name="c", subcore_axis_name="s",
                                  num_cores=2, num_subcores=_NSUB)


def _edge_body(h_hbm, src_hbm, dst_hbm, acc_out, idx_v, si_a, si_b, dst_v,
               rows_a, rows_b, acc_sh, sem_a, sem_b, sem_ia, sem_ib):
    c = lax.axis_index("c")
    s = lax.axis_index("s")
    wid = s * 2 + c
    ar = jnp.arange(16, dtype=jnp.int32)

    for k in range(_RPW // _CH):
        base = s * _RPW + k * _CH
        for q in range(_CH // 16):
            idx_v[pl.ds(q * 16, 16)] = ar + (base + q * 16)
        pltpu.sync_copy(h_hbm.at[pl.ds(base, _CH)], rows_a)
        pltpu.sync_copy(rows_a, acc_sh.at[idx_v])
    pltpu.sync_copy(dst_hbm.at[wid], dst_v)
    plsc.subcore_barrier()

    sbase = wid * _NCH
    pltpu.sync_copy(src_hbm.at[sbase], si_a)
    pltpu.async_copy(h_hbm.at[si_a], rows_a, sem_a)
    pltpu.async_copy(src_hbm.at[sbase + 1], si_b, sem_ib)

    def _pair(i, _):
        t = 2 * i
        pltpu.make_async_copy(src_hbm.at[sbase], si_b, sem_ib).wait()
        pltpu.async_copy(h_hbm.at[si_b], rows_b, sem_b)
        pltpu.make_async_copy(h_hbm.at[si_a], rows_a, sem_a).wait()
        pltpu.async_copy(src_hbm.at[sbase + t + 2], si_a, sem_ia)
        pltpu.sync_copy(rows_a, acc_sh.at[dst_v.at[t]], add=True)
        pltpu.make_async_copy(src_hbm.at[sbase], si_a, sem_ia).wait()
        pltpu.async_copy(h_hbm.at[si_a], rows_a, sem_a)
        pltpu.make_async_copy(h_hbm.at[si_b], rows_b, sem_b).wait()
        pltpu.async_copy(src_hbm.at[sbase + t + 3], si_b, sem_ib)
        pltpu.sync_copy(rows_b, acc_sh.at[dst_v.at[t + 1]], add=True)
        return 0

    lax.fori_loop(0, _NCH // 2 - 1, _pair, 0)
    t0 = _NCH - 2
    pltpu.make_async_copy(src_hbm.at[sbase], si_b, sem_ib).wait()
    pltpu.async_copy(h_hbm.at[si_b], rows_b, sem_b)
    pltpu.make_async_copy(h_hbm.at[si_a], rows_a, sem_a).wait()
    pltpu.sync_copy(rows_a, acc_sh.at[dst_v.at[t0]], add=True)
    pltpu.make_async_copy(h_hbm.at[si_b], rows_b, sem_b).wait()
    pltpu.sync_copy(rows_b, acc_sh.at[dst_v.at[t0 + 1]], add=True)
    plsc.subcore_barrier()

    for k in range(_RPW // _CH):
        base = s * _RPW + k * _CH
        for q in range(_CH // 16):
            idx_v[pl.ds(q * 16, 16)] = ar + (base + q * 16)
        pltpu.async_copy(acc_sh.at[idx_v], rows_a, sem_a).wait()
        pltpu.sync_copy(rows_a, acc_out.at[pl.ds(c * _N_PAD + base, _CH)])


@functools.cache
def _build_edge_kernel():
    return pl.kernel(
        _edge_body,
        out_type=jax.ShapeDtypeStruct((2 * _N_PAD, _H), jnp.float32),
        scratch_types=[
            pltpu.VMEM((_CH,), jnp.int32),
            pltpu.VMEM((_CH,), jnp.int32),
            pltpu.VMEM((_CH,), jnp.int32),
            pltpu.VMEM((_NCH, _CH), jnp.int32),
            pltpu.VMEM((_CH, _H), jnp.float32),
            pltpu.VMEM((_CH, _H), jnp.float32),
            pltpu.VMEM_SHARED((_N_PAD, _H), jnp.float32),
            pltpu.SemaphoreType.DMA,
            pltpu.SemaphoreType.DMA,
            pltpu.SemaphoreType.DMA,
            pltpu.SemaphoreType.DMA,
        ],
        mesh=_mesh(),
    )


def _edge_kernel(hp, srcp, dstp):
    return _build_edge_kernel()(hp, srcp, dstp)


def _hist_body(dst_ref, out_ref):
    i = pl.program_id(0)

    @pl.when(i == 0)
    def _init():
        out_ref[...] = jnp.zeros_like(out_ref)

    d = dst_ref[...]
    hi = d // 128
    lo = d - hi * 128
    oh_hi = (hi == lax.broadcasted_iota(jnp.int32, (1, _NHI), 1)
             ).astype(jnp.bfloat16)
    oh_lo = (lo == lax.broadcasted_iota(jnp.int32, (1, 128), 1)
             ).astype(jnp.bfloat16)
    out_ref[...] += lax.dot_general(oh_hi, oh_lo, (((0,), (0,)), ((), ())),
                                    preferred_element_type=jnp.float32)


def _hist(dst_col):
    return pl.pallas_call(
        _hist_body,
        grid=(_EGRID,),
        in_specs=[pl.BlockSpec((_EB, 1), lambda i: (i, 0))],
        out_specs=pl.BlockSpec((_NHI, 128), lambda i: (0, 0)),
        out_shape=jax.ShapeDtypeStruct((_NHI, 128), jnp.float32),
        compiler_params=pltpu.CompilerParams(
            dimension_semantics=("arbitrary",)),
    )(dst_col)


def _k2_body(x_ref, w_ref, deg_ref, h1p_ref, dinv_ref):
    h = jnp.dot(x_ref[...], w_ref[...], preferred_element_type=jnp.float32)
    dinv = lax.rsqrt(deg_ref[...] + 1.0)
    h1p_ref[...] = h * dinv
    dinv_ref[...] = dinv


def _k2(xp, W1, deg_col):
    return pl.pallas_call(
        _k2_body,
        grid=(_GRID,),
        in_specs=[
            pl.BlockSpec((_BLK, _D), lambda i: (i, 0)),
            pl.BlockSpec((_D, _H), lambda i: (0, 0)),
            pl.BlockSpec((_BLK, 1), lambda i: (i, 0)),
        ],
        out_specs=[
            pl.BlockSpec((_BLK, _H), lambda i: (i, 0)),
            pl.BlockSpec((_BLK, 1), lambda i: (i, 0)),
        ],
        out_shape=[
            jax.ShapeDtypeStruct((_N_PAD, _H), jnp.float32),
            jax.ShapeDtypeStruct((_N_PAD, 1), jnp.float32),
        ],
    )(xp, W1, deg_col)


def _k4_body(a_ref, hp_ref, dinv_ref, b_ref, w_ref, out_ref):
    a = a_ref[0] + a_ref[1]
    dinv = dinv_ref[...]
    g = jnp.maximum(dinv * (a - hp_ref[...]) + b_ref[...], 0.0)
    h = jnp.dot(g, w_ref[...], preferred_element_type=jnp.float32)
    out_ref[...] = h * dinv


def _k4(acc, h1p, dinv, b1, W2):
    return pl.pallas_call(
        _k4_body,
        grid=(_GRID,),
        in_specs=[
            pl.BlockSpec((2, _BLK, _H), lambda i: (0, i, 0)),
            pl.BlockSpec((_BLK, _H), lambda i: (i, 0)),
            pl.BlockSpec((_BLK, 1), lambda i: (i, 0)),
            pl.BlockSpec((1, _H), lambda i: (0, 0)),
            pl.BlockSpec((_H, _H), lambda i: (0, 0)),
        ],
        out_specs=pl.BlockSpec((_BLK, _H), lambda i: (i, 0)),
        out_shape=jax.ShapeDtypeStruct((_N_PAD, _H), jnp.float32),
    )(acc, h1p, dinv, b1, W2)


def _k6_body(a_ref, hp_ref, dinv_ref, b_ref, wlin_ref, blin_ref, batch_ref,
             out_ref, seg_acc, cnt_acc):
    i = pl.program_id(0)

    @pl.when(i == 0)
    def _init():
        seg_acc[...] = jnp.zeros_like(seg_acc)
        cnt_acc[...] = jnp.zeros_like(cnt_acc)

    a = a_ref[0] + a_ref[1]
    h = dinv_ref[...] * (a - hp_ref[...]) + b_ref[...]
    logits = jnp.dot(h, wlin_ref[...],
                     preferred_element_type=jnp.float32) + blin_ref[...]
    gids = lax.broadcasted_iota(jnp.int32, (1, _NG), 1)
    oh = (batch_ref[...] == gids).astype(jnp.float32)
    seg_acc[...] += lax.dot_general(oh, logits, (((0,), (0,)), ((), ())),
                                    preferred_element_type=jnp.float32)
    cnt_acc[...] += jnp.sum(oh, axis=0)[:, None]

    @pl.when(i == pl.num_programs(0) - 1)
    def _fin():
        pooled = seg_acc[...] / jnp.maximum(cnt_acc[...], 1.0)
        m = jnp.max(pooled, axis=1, keepdims=True)
        lse = jnp.log(jnp.sum(jnp.exp(pooled - m), axis=1, keepdims=True)) + m
        out_ref[...] = pooled - lse


def _k6(acc, h2p, dinv, b2, Wlin, blin, batchp):
    return pl.pallas_call(
        _k6_body,
        grid=(_GRID,),
        in_specs=[
            pl.BlockSpec((2, _BLK, _H), lambda i: (0, i, 0)),
            pl.BlockSpec((_BLK, _H), lambda i: (i, 0)),
            pl.BlockSpec((_BLK, 1), lambda i: (i, 0)),
            pl.BlockSpec((1, _H), lambda i: (0, 0)),
            pl.BlockSpec((_H, _NE), lambda i: (0, 0)),
            pl.BlockSpec((1, _NE), lambda i: (0, 0)),
            pl.BlockSpec((_BLK, 1), lambda i: (i, 0)),
        ],
        out_specs=pl.BlockSpec((_NG, _NE), lambda i: (0, 0)),
        out_shape=jax.ShapeDtypeStruct((_NG, _NE), jnp.float32),
        scratch_shapes=[
            pltpu.VMEM((_NG, _NE), jnp.float32),
            pltpu.VMEM((_NG, 1), jnp.float32),
        ],
        compiler_params=pltpu.CompilerParams(
            dimension_semantics=("arbitrary",)),
    )(acc, h2p, dinv, b2, Wlin, blin, batchp)


def kernel(x, edge_index, batch, W1, b1, W2, b2, Wlin, blin):
    pad_e = _E_PAD - _E
    fill = jnp.full((pad_e,), _N, jnp.int32)
    srcp = jnp.concatenate([edge_index[0], fill]).reshape(_NW * _NCH, _CH)
    dstp = jnp.concatenate([edge_index[1], fill]).reshape(_NW, _NCH, _CH)
    xp = jnp.zeros((_N_PAD, _D), jnp.float32).at[:_N].set(x)
    batchp = jnp.full((_N_PAD, 1), _NG, jnp.int32).at[:_N, 0].set(batch)

    deg_col = _hist(edge_index[1].reshape(_E, 1)).reshape(_N_PAD, 1)
    h1p, dinv = _k2(xp, W1, deg_col)
    acc1 = _edge_kernel(h1p, srcp, dstp).reshape(2, _N_PAD, _H)
    h2p = _k4(acc1, h1p, dinv, b1.reshape(1, _H), W2)
    acc2 = _edge_kernel(h2p, srcp, dstp).reshape(2, _N_PAD, _H)
    return _k6(acc2, h2p, dinv, b2.reshape(1, _H), Wlin, blin.reshape(1, _NE),
               batchp)


# speedup vs baseline: 1.2230x; 1.2230x over previous
import functools

import jax
import jax.numpy as jnp
from jax import lax
from jax.experimental import pallas as pl
from jax.experimental.pallas import tpu as pltpu
from jax.experimental.pallas import tpu_sc as plsc

_N = 10000
_E = 320000
_D = 128
_H = 128
_NE = 8
_NG = 64

_NW = 32
_NSUB = 16
_CH = 128
_NCH = 79
_E_PAD = _NW * _NCH * _CH
_N_PAD = 10240
_RPW = _N_PAD // _NSUB
_BLK = 1024
_GRID = _N_PAD // _BLK
_EB = 6400
_EGRID = _E // _EB
_NHI = _N_PAD // 128


@functools.cache
def _mesh():
    return plsc.VectorSubcoreMesh(core_axis_name="c", subcore_axis_name="s",
                                  num_cores=2, num_subcores=_NSUB)


def _edge_body(h_hbm, src_hbm, dst_hbm, acc_out, idx_v, src_v, dst_v, rows_v,
               acc_sh, sem):
    c = lax.axis_index("c")
    s = lax.axis_index("s")
    wid = s * 2 + c
    ar = jnp.arange(16, dtype=jnp.int32)

    for k in range(_RPW // _CH):
        base = s * _RPW + k * _CH
        for q in range(_CH // 16):
            idx_v[pl.ds(q * 16, 16)] = ar + (base + q * 16)
        pltpu.sync_copy(h_hbm.at[pl.ds(base, _CH)], rows_v)
        pltpu.sync_copy(rows_v, acc_sh.at[idx_v])
    pltpu.sync_copy(src_hbm.at[wid], src_v)
    pltpu.sync_copy(dst_hbm.at[wid], dst_v)
    plsc.subcore_barrier()

    def _chunk(j, _):
        pltpu.async_copy(h_hbm.at[src_v.at[j]], rows_v, sem).wait()
        pltpu.sync_copy(rows_v, acc_sh.at[dst_v.at[j]], add=True)
        return 0

    lax.fori_loop(0, _NCH, _chunk, 0)
    plsc.subcore_barrier()

    for k in range(_RPW // _CH):
        base = s * _RPW + k * _CH
        for q in range(_CH // 16):
            idx_v[pl.ds(q * 16, 16)] = ar + (base + q * 16)
        pltpu.async_copy(acc_sh.at[idx_v], rows_v, sem).wait()
        pltpu.sync_copy(rows_v, acc_out.at[pl.ds(c * _N_PAD + base, _CH)])


@functools.cache
def _build_edge_kernel():
    return pl.kernel(
        _edge_body,
        out_type=jax.ShapeDtypeStruct((2 * _N_PAD, _H), jnp.float32),
        scratch_types=[
            pltpu.VMEM((_CH,), jnp.int32),
            pltpu.VMEM((_NCH, _CH), jnp.int32),
            pltpu.VMEM((_NCH, _CH), jnp.int32),
            pltpu.VMEM((_CH, _H), jnp.float32),
            pltpu.VMEM_SHARED((_N_PAD, _H), jnp.float32),
            pltpu.SemaphoreType.DMA,
        ],
        mesh=_mesh(),
    )


def _edge_kernel(hp, srcp, dstp):
    return _build_edge_kernel()(hp, srcp, dstp)


def _hist_body(dst_ref, out_ref):
    i = pl.program_id(0)

    @pl.when(i == 0)
    def _init():
        out_ref[...] = jnp.zeros_like(out_ref)

    d = dst_ref[...]
    hi = d // 128
    lo = d - hi * 128
    oh_hi = (hi == lax.broadcasted_iota(jnp.int32, (1, _NHI), 1)
             ).astype(jnp.bfloat16)
    oh_lo = (lo == lax.broadcasted_iota(jnp.int32, (1, 128), 1)
             ).astype(jnp.bfloat16)
    out_ref[...] += lax.dot_general(oh_hi, oh_lo, (((0,), (0,)), ((), ())),
                                    preferred_element_type=jnp.float32)


def _hist(dst_col):
    return pl.pallas_call(
        _hist_body,
        grid=(_EGRID,),
        in_specs=[pl.BlockSpec((_EB, 1), lambda i: (i, 0))],
        out_specs=pl.BlockSpec((_NHI, 128), lambda i: (0, 0)),
        out_shape=jax.ShapeDtypeStruct((_NHI, 128), jnp.float32),
        compiler_params=pltpu.CompilerParams(
            dimension_semantics=("arbitrary",)),
    )(dst_col)


def _k2_body(x_ref, w_ref, deg_ref, h1p_ref, dinv_ref):
    h = jnp.dot(x_ref[...], w_ref[...], preferred_element_type=jnp.float32)
    dinv = lax.rsqrt(deg_ref[...] + 1.0)
    h1p_ref[...] = h * dinv
    dinv_ref[...] = dinv


def _k2(xp, W1, deg_col):
    return pl.pallas_call(
        _k2_body,
        grid=(_GRID,),
        in_specs=[
            pl.BlockSpec((_BLK, _D), lambda i: (i, 0)),
            pl.BlockSpec((_D, _H), lambda i: (0, 0)),
            pl.BlockSpec((_BLK, 1), lambda i: (i, 0)),
        ],
        out_specs=[
            pl.BlockSpec((_BLK, _H), lambda i: (i, 0)),
            pl.BlockSpec((_BLK, 1), lambda i: (i, 0)),
        ],
        out_shape=[
            jax.ShapeDtypeStruct((_N_PAD, _H), jnp.float32),
            jax.ShapeDtypeStruct((_N_PAD, 1), jnp.float32),
        ],
    )(xp, W1, deg_col)


def _k4_body(a_ref, hp_ref, dinv_ref, b_ref, w_ref, out_ref):
    a = a_ref[0] + a_ref[1]
    dinv = dinv_ref[...]
    g = jnp.maximum(dinv * (a - hp_ref[...]) + b_ref[...], 0.0)
    h = jnp.dot(g, w_ref[...], preferred_element_type=jnp.float32)
    out_ref[...] = h * dinv


def _k4(acc, h1p, dinv, b1, W2):
    return pl.pallas_call(
        _k4_body,
        grid=(_GRID,),
        in_specs=[
            pl.BlockSpec((2, _BLK, _H), lambda i: (0, i, 0)),
            pl.BlockSpec((_BLK, _H), lambda i: (i, 0)),
            pl.BlockSpec((_BLK, 1), lambda i: (i, 0)),
            pl.BlockSpec((1, _H), lambda i: (0, 0)),
            pl.BlockSpec((_H, _H), lambda i: (0, 0)),
        ],
        out_specs=pl.BlockSpec((_BLK, _H), lambda i: (i, 0)),
        out_shape=jax.ShapeDtypeStruct((_N_PAD, _H), jnp.float32),
    )(acc, h1p, dinv, b1, W2)


def _k6_body(a_ref, hp_ref, dinv_ref, b_ref, wlin_ref, blin_ref, batch_ref,
             out_ref, seg_acc, cnt_acc):
    i = pl.program_id(0)

    @pl.when(i == 0)
    def _init():
        seg_acc[...] = jnp.zeros_like(seg_acc)
        cnt_acc[...] = jnp.zeros_like(cnt_acc)

    a = a_ref[0] + a_ref[1]
    h = dinv_ref[...] * (a - hp_ref[...]) + b_ref[...]
    logits = jnp.dot(h, wlin_ref[...],
                     preferred_element_type=jnp.float32) + blin_ref[...]
    gids = lax.broadcasted_iota(jnp.int32, (1, _NG), 1)
    oh = (batch_ref[...] == gids).astype(jnp.float32)
    seg_acc[...] += lax.dot_general(oh, logits, (((0,), (0,)), ((), ())),
                                    preferred_element_type=jnp.float32)
    cnt_acc[...] += jnp.sum(oh, axis=0)[:, None]

    @pl.when(i == pl.num_programs(0) - 1)
    def _fin():
        pooled = seg_acc[...] / jnp.maximum(cnt_acc[...], 1.0)
        m = jnp.max(pooled, axis=1, keepdims=True)
        lse = jnp.log(jnp.sum(jnp.exp(pooled - m), axis=1, keepdims=True)) + m
        out_ref[...] = pooled - lse


def _k6(acc, h2p, dinv, b2, Wlin, blin, batchp):
    return pl.pallas_call(
        _k6_body,
        grid=(_GRID,),
        in_specs=[
            pl.BlockSpec((2, _BLK, _H), lambda i: (0, i, 0)),
            pl.BlockSpec((_BLK, _H), lambda i: (i, 0)),
            pl.BlockSpec((_BLK, 1), lambda i: (i, 0)),
            pl.BlockSpec((1, _H), lambda i: (0, 0)),
            pl.BlockSpec((_H, _NE), lambda i: (0, 0)),
            pl.BlockSpec((1, _NE), lambda i: (0, 0)),
            pl.BlockSpec((_BLK, 1), lambda i: (i, 0)),
        ],
        out_specs=pl.BlockSpec((_NG, _NE), lambda i: (0, 0)),
        out_shape=jax.ShapeDtypeStruct((_NG, _NE), jnp.float32),
        scratch_shapes=[
            pltpu.VMEM((_NG, _NE), jnp.float32),
            pltpu.VMEM((_NG, 1), jnp.float32),
        ],
        compiler_params=pltpu.CompilerParams(
            dimension_semantics=("arbitrary",)),
    )(acc, h2p, dinv, b2, Wlin, blin, batchp)


def kernel(x, edge_index, batch, W1, b1, W2, b2, Wlin, blin):
    pad_e = _E_PAD - _E
    fill = jnp.full((pad_e,), _N, jnp.int32)
    srcp = jnp.concatenate([edge_index[0], fill]).reshape(_NW, _NCH, _CH)
    dstp = jnp.concatenate([edge_index[1], fill]).reshape(_NW, _NCH, _CH)
    xp = jnp.zeros((_N_PAD, _D), jnp.float32).at[:_N].set(x)
    batchp = jnp.full((_N_PAD, 1), _NG, jnp.int32).at[:_N, 0].set(batch)

    deg_col = _hist(edge_index[1].reshape(_E, 1)).reshape(_N_PAD, 1)
    h1p, dinv = _k2(xp, W1, deg_col)
    acc1 = _edge_kernel(h1p, srcp, dstp).reshape(2, _N_PAD, _H)
    h2p = _k4(acc1, h1p, dinv, b1.reshape(1, _H), W2)
    acc2 = _edge_kernel(h2p, srcp, dstp).reshape(2, _N_PAD, _H)
    return _k6(acc2, h2p, dinv, b2.reshape(1, _H), Wlin, blin.reshape(1, _NE),
               batchp)


# speedup vs baseline: 1.7464x; 1.4279x over previous
import functools

import jax
import jax.numpy as jnp
from jax import lax
from jax.experimental import pallas as pl
from jax.experimental.pallas import tpu as pltpu
from jax.experimental.pallas import tpu_sc as plsc

_N = 10000
_E = 320000
_D = 128
_H = 128
_NE = 8
_NG = 64

_NW = 32
_NSUB = 16
_CH = 128
_NCH = 79
_E_PAD = _NW * _NCH * _CH
_N_PAD = 10240
_RPW = _N_PAD // _NSUB
_BLK = 1024
_GRID = _N_PAD // _BLK
_EB = 6400
_EGRID = _E // _EB
_NHI = _N_PAD // 128


@functools.cache
def _mesh():
    return plsc.VectorSubcoreMesh(core_axis_name="c", subcore_axis_name="s",
                                  num_cores=2, num_subcores=_NSUB)


def _edge_body(h_hbm, src_hbm, dst_hbm, acc_out, idx_v, src_v, dst_v, rows_v,
               acc_sh, sem):
    c = lax.axis_index("c")
    s = lax.axis_index("s")
    wid = s * 2 + c
    ar = jnp.arange(16, dtype=jnp.int32)

    for k in range(_RPW // _CH):
        base = s * _RPW + k * _CH
        for q in range(_CH // 16):
            idx_v[pl.ds(q * 16, 16)] = ar + (base + q * 16)
        pltpu.sync_copy(h_hbm.at[pl.ds(base, _CH)], rows_v)
        pltpu.sync_copy(rows_v, acc_sh.at[idx_v])
    pltpu.sync_copy(src_hbm.at[wid], src_v)
    pltpu.sync_copy(dst_hbm.at[wid], dst_v)
    plsc.subcore_barrier()

    def _chunk(j, _):
        pltpu.async_copy(h_hbm.at[src_v.at[j]], rows_v, sem).wait()
        pltpu.sync_copy(rows_v, acc_sh.at[dst_v.at[j]], add=True)
        return 0

    lax.fori_loop(0, _NCH, _chunk, 0)
    plsc.subcore_barrier()

    for k in range(_RPW // _CH):
        base = s * _RPW + k * _CH
        for q in range(_CH // 16):
            idx_v[pl.ds(q * 16, 16)] = ar + (base + q * 16)
        pltpu.async_copy(acc_sh.at[idx_v], rows_v, sem).wait()
        pltpu.sync_copy(rows_v, acc_out.at[pl.ds(c * _N_PAD + base, _CH)])


@functools.cache
def _build_edge_kernel():
    return pl.kernel(
        _edge_body,
        out_type=jax.ShapeDtypeStruct((2 * _N_PAD, _H), jnp.float32),
        scratch_types=[
            pltpu.VMEM((_CH,), jnp.int32),
            pltpu.VMEM((_NCH, _CH), jnp.int32),
            pltpu.VMEM((_NCH, _CH), jnp.int32),
            pltpu.VMEM((_CH, _H), jnp.float32),
            pltpu.VMEM_SHARED((_N_PAD, _H), jnp.float32),
            pltpu.SemaphoreType.DMA,
        ],
        mesh=_mesh(),
    )


def _edge_kernel(hp, srcp, dstp):
    return _build_edge_kernel()(hp, srcp, dstp)


def _hist_body(dst_ref, out_ref):
    i = pl.program_id(0)

    @pl.when(i == 0)
    def _init():
        out_ref[...] = jnp.zeros_like(out_ref)

    d = dst_ref[...]
    hi = d // 128
    lo = d - hi * 128
    oh_hi = (hi == lax.broadcasted_iota(jnp.int32, (1, _NHI), 1)
             ).astype(jnp.bfloat16)
    oh_lo = (lo == lax.broadcasted_iota(jnp.int32, (1, 128), 1)
             ).astype(jnp.bfloat16)
    out_ref[...] += lax.dot_general(oh_hi, oh_lo, (((0,), (0,)), ((), ())),
                                    preferred_element_type=jnp.float32)


def _hist(dst_col):
    return pl.pallas_call(
        _hist_body,
        grid=(_EGRID,),
        in_specs=[pl.BlockSpec((_EB, 1), lambda i: (i, 0))],
        out_specs=pl.BlockSpec((_NHI, 128), lambda i: (0, 0)),
        out_shape=jax.ShapeDtypeStruct((_NHI, 128), jnp.float32),
        compiler_params=pltpu.CompilerParams(
            dimension_semantics=("arbitrary",)),
    )(dst_col)


def _k2_body(x_ref, w_ref, deg_ref, h1p_ref, dinv_ref):
    h = jnp.dot(x_ref[...], w_ref[...], preferred_element_type=jnp.float32)
    dinv = lax.rsqrt(deg_ref[...] + 1.0)
    h1p_ref[...] = h * dinv
    dinv_ref[...] = dinv


def _k2(xp, W1, deg_col):
    return pl.pallas_call(
        _k2_body,
        grid=(_GRID,),
        in_specs=[
            pl.BlockSpec((_BLK, _D), lambda i: (i, 0)),
            pl.BlockSpec((_D, _H), lambda i: (0, 0)),
            pl.BlockSpec((_BLK, 1), lambda i: (i, 0)),
        ],
        out_specs=[
            pl.BlockSpec((_BLK, _H), lambda i: (i, 0)),
            pl.BlockSpec((_BLK, 1), lambda i: (i, 0)),
        ],
        out_shape=[
            jax.ShapeDtypeStruct((_N_PAD, _H), jnp.float32),
            jax.ShapeDtypeStruct((_N_PAD, 1), jnp.float32),
        ],
    )(xp, W1, deg_col)


def _k4_body(a_ref, hp_ref, dinv_ref, b_ref, w_ref, out_ref):
    a = a_ref[0] + a_ref[1]
    dinv = dinv_ref[...]
    g = jnp.maximum(dinv * (a - hp_ref[...]) + b_ref[...], 0.0)
    h = jnp.dot(g, w_ref[...], preferred_element_type=jnp.float32)
    out_ref[...] = h * dinv


def _k4(acc, h1p, dinv, b1, W2):
    return pl.pallas_call(
        _k4_body,
        grid=(_GRID,),
        in_specs=[
            pl.BlockSpec((2, _BLK, _H), lambda i: (0, i, 0)),
            pl.BlockSpec((_BLK, _H), lambda i: (i, 0)),
            pl.BlockSpec((_BLK, 1), lambda i: (i, 0)),
            pl.BlockSpec((1, _H), lambda i: (0, 0)),
            pl.BlockSpec((_H, _H), lambda i: (0, 0)),
        ],
        out_specs=pl.BlockSpec((_BLK, _H), lambda i: (i, 0)),
        out_shape=jax.ShapeDtypeStruct((_N_PAD, _H), jnp.float32),
    )(acc, h1p, dinv, b1, W2)


def _k6_body(a_ref, hp_ref, dinv_ref, b_ref, wlin_ref, blin_ref, batch_ref,
             out_ref, seg_acc, cnt_acc):
    i = pl.program_id(0)

    @pl.when(i == 0)
    def _init():
        seg_acc[...] = jnp.zeros_like(seg_acc)
        cnt_acc[...] = jnp.zeros_like(cnt_acc)

    a = a_ref[0] + a_ref[1]
    h = dinv_ref[...] * (a - hp_ref[...]) + b_ref[...]
    logits = jnp.dot(h, wlin_ref[...],
                     preferred_element_type=jnp.float32) + blin_ref[...]
    gids = lax.broadcasted_iota(jnp.int32, (1, _NG), 1)
    oh = (batch_ref[...] == gids).astype(jnp.float32)
    seg_acc[...] += lax.dot_general(oh, logits, (((0,), (0,)), ((), ())),
                                    preferred_element_type=jnp.float32)
    cnt_acc[...] += jnp.sum(oh, axis=0)[:, None]

    @pl.when(i == pl.num_programs(0) - 1)
    def _fin():
        pooled = seg_acc[...] / jnp.maximum(cnt_acc[...], 1.0)
        m = jnp.max(pooled, axis=1, keepdims=True)
        lse = jnp.log(jnp.sum(jnp.exp(pooled - m), axis=1, keepdims=True)) + m
        out_ref[...] = pooled - lse


def _k6(acc, h2p, dinv, b2, Wlin, blin, batchp):
    return pl.pallas_call(
        _k6_body,
        grid=(_GRID,),
        in_specs=[
            pl.BlockSpec((2, _BLK, _H), lambda i: (0, i, 0)),
            pl.BlockSpec((_BLK, _H), lambda i: (i, 0)),
            pl.BlockSpec((_BLK, 1), lambda i: (i, 0)),
            pl.BlockSpec((1, _H), lambda i: (0, 0)),
            pl.BlockSpec((_H, _NE), lambda i: (0, 0)),
            pl.BlockSpec((1, _NE), lambda i: (0, 0)),
            pl.BlockSpec((_BLK, 1), lambda i: (i, 0)),
        ],
        out_specs=pl.BlockSpec((_NG, _NE), lambda i: (0, 0)),
        out_shape=jax.ShapeDtypeStruct((_NG, _NE), jnp.float32),
        scratch_shapes=[
            pltpu.VMEM((_NG, _NE), jnp.float32),
            pltpu.VMEM((_NG, 1), jnp.float32),
        ],
        compiler_params=pltpu.CompilerParams(
            dimension_semantics=("arbitrary",)),
    )(acc, h2p, dinv, b2, Wlin, blin, batchp)


def kernel(x, edge_index, batch, W1, b1, W2, b2, Wlin, blin):
    pad_e = _E_PAD - _E
    fill = _N + (jnp.arange(pad_e, dtype=jnp.int32) % (_N_PAD - _N))
    srcp = jnp.concatenate([edge_index[0], fill]).reshape(_NW, _NCH, _CH)
    dstp = jnp.concatenate([edge_index[1], fill]).reshape(_NW, _NCH, _CH)
    xp = jnp.zeros((_N_PAD, _D), jnp.float32).at[:_N].set(x)
    batchp = jnp.full((_N_PAD, 1), _NG, jnp.int32).at[:_N, 0].set(batch)

    deg_col = _hist(edge_index[1].reshape(_E, 1)).reshape(_N_PAD, 1)
    h1p, dinv = _k2(xp, W1, deg_col)
    acc1 = _edge_kernel(h1p, srcp, dstp).reshape(2, _N_PAD, _H)
    h2p = _k4(acc1, h1p, dinv, b1.reshape(1, _H), W2)
    acc2 = _edge_kernel(h2p, srcp, dstp).reshape(2, _N_PAD, _H)
    return _k6(acc2, h2p, dinv, b2.reshape(1, _H), Wlin, blin.reshape(1, _NE),
               batchp)
